# Initial kernel scaffold; baseline (speedup 1.0000x reference)
import jax
import jax.numpy as jnp
from jax.experimental import pallas as pl


def kernel(graph, data, W1, b1, W2, b2):
    raise NotImplementedError("write your pallas kernel here")


# speedup vs baseline: 16.9789x; 16.9789x over previous
import functools

import jax
import jax.numpy as jnp
from jax import lax
from jax.experimental import pallas as pl
from jax.experimental.pallas import tpu as pltpu
from jax.experimental.pallas import tpu_sc as plsc

N = 10000
E = 160000
IN = 256
H = 512
C = 128

@functools.lru_cache(maxsize=None)
def _mesh():
    return plsc.VectorSubcoreMesh(core_axis_name="c", subcore_axis_name="s")


def _fill_zeros(ref, rows, width):
    def body(i, _):
        def inner(j, __):
            ref[i, pl.ds(j * 16, 16)] = jnp.zeros((16,), jnp.float32)
            return 0
        return lax.fori_loop(0, width // 16, inner, 0)
    lax.fori_loop(0, rows, body, 0)


_E_PER_SC = E // 2
_E_PER_TILE_DEG = _E_PER_SC // 16
_KDEG = 1000


@functools.lru_cache(maxsize=None)
def _make_deg_kernel():
    return functools.partial(
        pl.kernel,
        mesh=_mesh(),
        out_type=[
            jax.ShapeDtypeStruct((N,), jnp.float32),
            jax.ShapeDtypeStruct((N,), jnp.float32),
        ],
        scratch_types=[
            pltpu.VMEM((1008,), jnp.float32),
            pltpu.VMEM((_KDEG,), jnp.int32),
            pltpu.VMEM((1000,), jnp.float32),
            pltpu.VMEM_SHARED((N,), jnp.float32),
        ],
    )(_deg_body)


def _deg_body(dst_hbm, out0, out1, ones_v, idx_v, bounce, acc):
    c = lax.axis_index("c")
    s = lax.axis_index("s")

    def fill(j, _):
        ones_v[pl.ds(j * 16, 16)] = jnp.full((16,), 1.0, jnp.float32)
        return 0
    lax.fori_loop(0, 63, fill, 0)

    @pl.when(s < 10)
    def _():
        def zfill(j, _):
            bounce[pl.ds(j * 16, 16)] = jnp.zeros((16,), jnp.float32)
            return 0
        lax.fori_loop(0, 1000 // 16, zfill, 0)
        pltpu.sync_copy(bounce, acc.at[pl.ds(pl.multiple_of(s * 1000, 8), 1000)])

    plsc.subcore_barrier()

    base_edge = c * _E_PER_SC + s * _E_PER_TILE_DEG
    for j in range(_E_PER_TILE_DEG // _KDEG):
        off = pl.multiple_of(base_edge + j * _KDEG, 8)
        pltpu.sync_copy(dst_hbm.at[pl.ds(off, _KDEG)], idx_v)
        pltpu.sync_copy(ones_v.at[pl.ds(0, _KDEG)], acc.at[idx_v], add=True)

    plsc.subcore_barrier()

    @pl.when(s < 10)
    def _():
        o = pl.multiple_of(s * 1000, 8)
        pltpu.sync_copy(acc.at[pl.ds(o, 1000)], bounce)

        @pl.when(c == 0)
        def _():
            pltpu.sync_copy(bounce, out0.at[pl.ds(o, 1000)])

        @pl.when(c == 1)
        def _():
            pltpu.sync_copy(bounce, out1.at[pl.ds(o, 1000)])


_E_PER_TILE = E // 16
_ZR = 200


_F = 128


@functools.lru_cache(maxsize=None)
def _make_edge_sum(split_features, K):
    def edge_sum(src_hbm, dst_hbm, xa_hbm, xb_hbm, oa_hbm, ob_hbm,
                 idx_s, idx_d, rows, acc):
        c = lax.axis_index("c")
        s = lax.axis_index("s")

        _fill_zeros(rows, _ZR, _F)

        @pl.when(s < 10)
        def _():
            for i in range(5):
                r0 = pl.multiple_of(s * 1000 + i * _ZR, 8)
                pltpu.sync_copy(rows.at[pl.ds(0, _ZR), :],
                                acc.at[pl.ds(r0, _ZR), :])

        plsc.subcore_barrier()

        if split_features:
            e_per_tile = E // 16
            base_edge = s * e_per_tile
        else:
            e_per_tile = E // 32
            base_edge = c * (E // 2) + s * e_per_tile
        for j in range(e_per_tile // K):
            e0 = pl.multiple_of(base_edge + j * K, 8)
            pltpu.sync_copy(src_hbm.at[pl.ds(e0, K)], idx_s)
            pltpu.sync_copy(dst_hbm.at[pl.ds(e0, K)], idx_d)

            if split_features:
                @pl.when(c == 0)
                def _():
                    pltpu.sync_copy(xa_hbm.at[idx_s], rows)

                @pl.when(c == 1)
                def _():
                    pltpu.sync_copy(xb_hbm.at[idx_s], rows)
            else:
                pltpu.sync_copy(xa_hbm.at[idx_s], rows)

            pltpu.sync_copy(rows, acc.at[idx_d], add=True)

        plsc.subcore_barrier()

        @pl.when(s < 10)
        def _():
            for i in range(5):
                r0 = pl.multiple_of(s * 1000 + i * _ZR, 8)
                pltpu.sync_copy(acc.at[pl.ds(r0, _ZR), :],
                                rows.at[pl.ds(0, _ZR), :])

                @pl.when(c == 0)
                def _():
                    pltpu.sync_copy(rows.at[pl.ds(0, _ZR), :],
                                    oa_hbm.at[pl.ds(r0, _ZR), :])

                @pl.when(c == 1)
                def _():
                    pltpu.sync_copy(rows.at[pl.ds(0, _ZR), :],
                                    ob_hbm.at[pl.ds(r0, _ZR), :])

    return functools.partial(
        pl.kernel,
        mesh=_mesh(),
        out_type=[
            jax.ShapeDtypeStruct((N, _F), jnp.float32),
            jax.ShapeDtypeStruct((N, _F), jnp.float32),
        ],
        scratch_types=[
            pltpu.VMEM((K,), jnp.int32),
            pltpu.VMEM((K,), jnp.int32),
            pltpu.VMEM((K, _F), jnp.float32),
            pltpu.VMEM_SHARED((N, _F), jnp.float32),
        ],
    )(edge_sum)


_BROW = 1000


def _scale_split_body(deg0_ref, deg1_ref, x_ref, dinv_ref, xa_ref, xb_ref):
    deg = deg0_ref[...] + deg1_ref[...] + 1.0
    dinv = lax.rsqrt(deg)
    dinv_ref[...] = dinv
    xs = x_ref[...] * dinv
    xa_ref[...] = xs[:, : IN // 2]
    xb_ref[...] = xs[:, IN // 2:]


def _scale_split(deg0, deg1, x):
    return pl.pallas_call(
        _scale_split_body,
        grid=(N // _BROW,),
        in_specs=[
            pl.BlockSpec((_BROW, 1), lambda i: (i, 0)),
            pl.BlockSpec((_BROW, 1), lambda i: (i, 0)),
            pl.BlockSpec((_BROW, IN), lambda i: (i, 0)),
        ],
        out_specs=[
            pl.BlockSpec((_BROW, 1), lambda i: (i, 0)),
            pl.BlockSpec((_BROW, IN // 2), lambda i: (i, 0)),
            pl.BlockSpec((_BROW, IN // 2), lambda i: (i, 0)),
        ],
        out_shape=[
            jax.ShapeDtypeStruct((N, 1), jnp.float32),
            jax.ShapeDtypeStruct((N, IN // 2), jnp.float32),
            jax.ShapeDtypeStruct((N, IN // 2), jnp.float32),
        ],
    )(deg0.reshape(N, 1), deg1.reshape(N, 1), x)


def _dense_body(sa_ref, sb_ref, x_ref, dinv_ref, w1_ref, b1_ref, w2_ref,
                gs_ref):
    dinv = dinv_ref[...]
    s1 = jnp.concatenate([sa_ref[...], sb_ref[...]], axis=1)
    z1 = dinv * s1 + (dinv * dinv) * x_ref[...]
    h1 = jnp.maximum(
        jnp.dot(z1, w1_ref[...], preferred_element_type=jnp.float32)
        + b1_ref[...], 0.0)
    g = jnp.dot(h1, w2_ref[...], preferred_element_type=jnp.float32)
    gs_ref[...] = dinv * g


def _dense(sa, sb, x, dinv, W1, b1, W2):
    return pl.pallas_call(
        _dense_body,
        grid=(N // _BROW,),
        in_specs=[
            pl.BlockSpec((_BROW, IN // 2), lambda i: (i, 0)),
            pl.BlockSpec((_BROW, IN // 2), lambda i: (i, 0)),
            pl.BlockSpec((_BROW, IN), lambda i: (i, 0)),
            pl.BlockSpec((_BROW, 1), lambda i: (i, 0)),
            pl.BlockSpec((IN, H), lambda i: (0, 0)),
            pl.BlockSpec((1, H), lambda i: (0, 0)),
            pl.BlockSpec((H, C), lambda i: (0, 0)),
        ],
        out_specs=pl.BlockSpec((_BROW, C), lambda i: (i, 0)),
        out_shape=jax.ShapeDtypeStruct((N, C), jnp.float32),
    )(sa, sb, x, dinv, W1, b1.reshape(1, H), W2)


def _final_body(t0_ref, t1_ref, gs_ref, dinv_ref, b2_ref, out_ref):
    dinv = dinv_ref[...]
    z2 = dinv * (t0_ref[...] + t1_ref[...] + gs_ref[...]) + b2_ref[...]
    m = jnp.max(z2, axis=1, keepdims=True)
    ez = jnp.exp(z2 - m)
    lse = jnp.log(jnp.sum(ez, axis=1, keepdims=True)) + m
    out_ref[...] = z2 - lse


def _final(t0, t1, gs, dinv, b2):
    return pl.pallas_call(
        _final_body,
        grid=(N // _BROW,),
        in_specs=[
            pl.BlockSpec((_BROW, C), lambda i: (i, 0)),
            pl.BlockSpec((_BROW, C), lambda i: (i, 0)),
            pl.BlockSpec((_BROW, C), lambda i: (i, 0)),
            pl.BlockSpec((_BROW, 1), lambda i: (i, 0)),
            pl.BlockSpec((1, C), lambda i: (0, 0)),
        ],
        out_specs=pl.BlockSpec((_BROW, C), lambda i: (i, 0)),
        out_shape=jax.ShapeDtypeStruct((N, C), jnp.float32),
    )(t0, t1, gs, dinv, b2.reshape(1, C))


def kernel(graph, data, W1, b1, W2, b2):
    src = graph[0]
    dst = graph[1]
    deg0, deg1 = _make_deg_kernel()(dst)
    dinv, xa, xb = _scale_split(deg0, deg1, data)
    sa, sb = _make_edge_sum(True, 200)(src, dst, xa, xb)
    gs = _dense(sa, sb, data, dinv, W1, b1, W2)
    t0, t1 = _make_edge_sum(False, 200)(src, dst, gs, gs)
    return _final(t0, t1, gs, dinv, b2)


# speedup vs baseline: 24.3109x; 1.4318x over previous
import functools

import jax
import jax.numpy as jnp
from jax import lax
from jax.experimental import pallas as pl
from jax.experimental.pallas import tpu as pltpu
from jax.experimental.pallas import tpu_sc as plsc

N = 10000
E = 160000
IN = 256
H = 512
C = 128

@functools.lru_cache(maxsize=None)
def _mesh():
    return plsc.VectorSubcoreMesh(core_axis_name="c", subcore_axis_name="s")


def _fill_zeros(ref, rows, width):
    def body(i, _):
        def inner(j, __):
            ref[i, pl.ds(j * 16, 16)] = jnp.zeros((16,), jnp.float32)
            return 0
        return lax.fori_loop(0, width // 16, inner, 0)
    lax.fori_loop(0, rows, body, 0)


_E_PER_SC = E // 2
_E_PER_TILE_DEG = _E_PER_SC // 16
_KDEG = 1000


@functools.lru_cache(maxsize=None)
def _make_deg_kernel():
    return functools.partial(
        pl.kernel,
        mesh=_mesh(),
        out_type=[
            jax.ShapeDtypeStruct((N,), jnp.float32),
            jax.ShapeDtypeStruct((N,), jnp.float32),
        ],
        scratch_types=[
            pltpu.VMEM((1008,), jnp.float32),
            pltpu.VMEM((_KDEG,), jnp.int32),
            pltpu.VMEM((1000,), jnp.float32),
            pltpu.VMEM_SHARED((N,), jnp.float32),
        ],
    )(_deg_body)


def _deg_body(dst_hbm, out0, out1, ones_v, idx_v, bounce, acc):
    c = lax.axis_index("c")
    s = lax.axis_index("s")

    def fill(j, _):
        ones_v[pl.ds(j * 16, 16)] = jnp.full((16,), 1.0, jnp.float32)
        return 0
    lax.fori_loop(0, 63, fill, 0)

    @pl.when(s < 10)
    def _():
        def zfill(j, _):
            bounce[pl.ds(j * 16, 16)] = jnp.zeros((16,), jnp.float32)
            return 0
        lax.fori_loop(0, 1000 // 16, zfill, 0)
        pltpu.sync_copy(bounce, acc.at[pl.ds(pl.multiple_of(s * 1000, 8), 1000)])

    plsc.subcore_barrier()

    base_edge = c * _E_PER_SC + s * _E_PER_TILE_DEG
    for j in range(_E_PER_TILE_DEG // _KDEG):
        off = pl.multiple_of(base_edge + j * _KDEG, 8)
        pltpu.sync_copy(dst_hbm.at[pl.ds(off, _KDEG)], idx_v)
        pltpu.sync_copy(ones_v.at[pl.ds(0, _KDEG)], acc.at[idx_v], add=True)

    plsc.subcore_barrier()

    @pl.when(s < 10)
    def _():
        o = pl.multiple_of(s * 1000, 8)
        pltpu.sync_copy(acc.at[pl.ds(o, 1000)], bounce)

        @pl.when(c == 0)
        def _():
            pltpu.sync_copy(bounce, out0.at[pl.ds(o, 1000)])

        @pl.when(c == 1)
        def _():
            pltpu.sync_copy(bounce, out1.at[pl.ds(o, 1000)])


_E_PER_TILE = E // 16
_ZR = 200


_F = 128


@functools.lru_cache(maxsize=None)
def _make_edge_sum(split_features, K):
    if split_features:
        e_per_tile = E // 16
    else:
        e_per_tile = E // 32
    J = e_per_tile // K
    T = e_per_tile - J * K
    wb = []
    r = 0
    while r < 1000:
        blk = min(K, 1000 - r)
        wb.append((r, blk))
        r += blk

    def edge_sum(src_hbm, dst_hbm, xa_hbm, xb_hbm, oa_hbm, ob_hbm,
                 is0, is1, id0, id1, rw0, rw1, acc, si0, si1, sg0, sg1):
        c = lax.axis_index("c")
        s = lax.axis_index("s")
        ibs = (is0, is1)
        ibd = (id0, id1)
        rbs = (rw0, rw1)
        sis = (si0, si1)
        sgs = (sg0, sg1)

        _fill_zeros(rw0, K, _F)

        @pl.when(s < 10)
        def _():
            for (r0, blk) in wb:
                ro = pl.multiple_of(s * 1000 + r0, 8)
                pltpu.sync_copy(rw0.at[pl.ds(0, blk), :],
                                acc.at[pl.ds(ro, blk), :])

        plsc.subcore_barrier()

        if split_features:
            base_edge = s * e_per_tile
        else:
            base_edge = c * (E // 2) + s * e_per_tile

        def eoff(j):
            return pl.multiple_of(base_edge + j * K, 8)

        def idx_start(j, b):
            pltpu.async_copy(src_hbm.at[pl.ds(eoff(j), K)], ibs[b], sis[b])
            pltpu.async_copy(dst_hbm.at[pl.ds(eoff(j), K)], ibd[b], sis[b])

        def idx_wait(j, b):
            pltpu.make_async_copy(
                src_hbm.at[pl.ds(eoff(j), K)], ibs[b], sis[b]).wait()
            pltpu.make_async_copy(
                dst_hbm.at[pl.ds(eoff(j), K)], ibd[b], sis[b]).wait()

        def gather_start(b):
            if split_features:
                @pl.when(c == 0)
                def _():
                    pltpu.async_copy(xa_hbm.at[ibs[b]], rbs[b], sgs[b])

                @pl.when(c == 1)
                def _():
                    pltpu.async_copy(xb_hbm.at[ibs[b]], rbs[b], sgs[b])
            else:
                pltpu.async_copy(xa_hbm.at[ibs[b]], rbs[b], sgs[b])

        def gather_wait(b):
            pltpu.make_async_copy(xa_hbm.at[ibs[b]], rbs[b], sgs[b]).wait()

        idx_start(0, 0)
        idx_wait(0, 0)
        gather_start(0)
        if J > 1:
            idx_start(1, 1)
        for j in range(J):
            b = j % 2
            nb = 1 - b
            gather_wait(b)
            if j + 1 < J:
                idx_wait(j + 1, nb)
                gather_start(nb)
            pltpu.sync_copy(rbs[b], acc.at[ibd[b]], add=True)
            if j + 2 < J:
                idx_start(j + 2, b)

        if T:
            t0 = pl.multiple_of(base_edge + J * K, 8)
            pltpu.sync_copy(src_hbm.at[pl.ds(t0, T)], is0.at[pl.ds(0, T)])
            pltpu.sync_copy(dst_hbm.at[pl.ds(t0, T)], id0.at[pl.ds(0, T)])
            if not split_features:
                pltpu.sync_copy(xa_hbm.at[is0.at[pl.ds(0, T)]],
                                rw0.at[pl.ds(0, T), :])
            else:
                @pl.when(c == 0)
                def _():
                    pltpu.sync_copy(xa_hbm.at[is0.at[pl.ds(0, T)]],
                                    rw0.at[pl.ds(0, T), :])

                @pl.when(c == 1)
                def _():
                    pltpu.sync_copy(xb_hbm.at[is0.at[pl.ds(0, T)]],
                                    rw0.at[pl.ds(0, T), :])
            pltpu.sync_copy(rw0.at[pl.ds(0, T), :],
                            acc.at[id0.at[pl.ds(0, T)]], add=True)

        plsc.subcore_barrier()

        @pl.when(s < 10)
        def _():
            for (r0, blk) in wb:
                ro = pl.multiple_of(s * 1000 + r0, 8)
                pltpu.sync_copy(acc.at[pl.ds(ro, blk), :],
                                rw0.at[pl.ds(0, blk), :])

                @pl.when(c == 0)
                def _():
                    pltpu.sync_copy(rw0.at[pl.ds(0, blk), :],
                                    oa_hbm.at[pl.ds(ro, blk), :])

                @pl.when(c == 1)
                def _():
                    pltpu.sync_copy(rw0.at[pl.ds(0, blk), :],
                                    ob_hbm.at[pl.ds(ro, blk), :])

    return functools.partial(
        pl.kernel,
        mesh=_mesh(),
        out_type=[
            jax.ShapeDtypeStruct((N, _F), jnp.float32),
            jax.ShapeDtypeStruct((N, _F), jnp.float32),
        ],
        scratch_types=[
            pltpu.VMEM((K,), jnp.int32),
            pltpu.VMEM((K,), jnp.int32),
            pltpu.VMEM((K,), jnp.int32),
            pltpu.VMEM((K,), jnp.int32),
            pltpu.VMEM((K, _F), jnp.float32),
            pltpu.VMEM((K, _F), jnp.float32),
            pltpu.VMEM_SHARED((N, _F), jnp.float32),
            pltpu.SemaphoreType.DMA,
            pltpu.SemaphoreType.DMA,
            pltpu.SemaphoreType.DMA,
            pltpu.SemaphoreType.DMA,
        ],
    )(edge_sum)


_BROW = 1000


def _scale_split_body(deg0_ref, deg1_ref, x_ref, dinv_ref, xa_ref, xb_ref):
    deg = deg0_ref[...] + deg1_ref[...] + 1.0
    dinv = lax.rsqrt(deg)
    dinv_ref[...] = dinv
    xs = x_ref[...] * dinv
    xa_ref[...] = xs[:, : IN // 2]
    xb_ref[...] = xs[:, IN // 2:]


def _scale_split(deg0, deg1, x):
    return pl.pallas_call(
        _scale_split_body,
        grid=(N // _BROW,),
        in_specs=[
            pl.BlockSpec((_BROW, 1), lambda i: (i, 0)),
            pl.BlockSpec((_BROW, 1), lambda i: (i, 0)),
            pl.BlockSpec((_BROW, IN), lambda i: (i, 0)),
        ],
        out_specs=[
            pl.BlockSpec((_BROW, 1), lambda i: (i, 0)),
            pl.BlockSpec((_BROW, IN // 2), lambda i: (i, 0)),
            pl.BlockSpec((_BROW, IN // 2), lambda i: (i, 0)),
        ],
        out_shape=[
            jax.ShapeDtypeStruct((N, 1), jnp.float32),
            jax.ShapeDtypeStruct((N, IN // 2), jnp.float32),
            jax.ShapeDtypeStruct((N, IN // 2), jnp.float32),
        ],
    )(deg0.reshape(N, 1), deg1.reshape(N, 1), x)


def _dense_body(sa_ref, sb_ref, x_ref, dinv_ref, w1_ref, b1_ref, w2_ref,
                gs_ref):
    dinv = dinv_ref[...]
    s1 = jnp.concatenate([sa_ref[...], sb_ref[...]], axis=1)
    z1 = dinv * s1 + (dinv * dinv) * x_ref[...]
    h1 = jnp.maximum(
        jnp.dot(z1, w1_ref[...], preferred_element_type=jnp.float32)
        + b1_ref[...], 0.0)
    g = jnp.dot(h1, w2_ref[...], preferred_element_type=jnp.float32)
    gs_ref[...] = dinv * g


def _dense(sa, sb, x, dinv, W1, b1, W2):
    return pl.pallas_call(
        _dense_body,
        grid=(N // _BROW,),
        in_specs=[
            pl.BlockSpec((_BROW, IN // 2), lambda i: (i, 0)),
            pl.BlockSpec((_BROW, IN // 2), lambda i: (i, 0)),
            pl.BlockSpec((_BROW, IN), lambda i: (i, 0)),
            pl.BlockSpec((_BROW, 1), lambda i: (i, 0)),
            pl.BlockSpec((IN, H), lambda i: (0, 0)),
            pl.BlockSpec((1, H), lambda i: (0, 0)),
            pl.BlockSpec((H, C), lambda i: (0, 0)),
        ],
        out_specs=pl.BlockSpec((_BROW, C), lambda i: (i, 0)),
        out_shape=jax.ShapeDtypeStruct((N, C), jnp.float32),
    )(sa, sb, x, dinv, W1, b1.reshape(1, H), W2)


def _final_body(t0_ref, t1_ref, gs_ref, dinv_ref, b2_ref, out_ref):
    dinv = dinv_ref[...]
    z2 = dinv * (t0_ref[...] + t1_ref[...] + gs_ref[...]) + b2_ref[...]
    m = jnp.max(z2, axis=1, keepdims=True)
    ez = jnp.exp(z2 - m)
    lse = jnp.log(jnp.sum(ez, axis=1, keepdims=True)) + m
    out_ref[...] = z2 - lse


def _final(t0, t1, gs, dinv, b2):
    return pl.pallas_call(
        _final_body,
        grid=(N // _BROW,),
        in_specs=[
            pl.BlockSpec((_BROW, C), lambda i: (i, 0)),
            pl.BlockSpec((_BROW, C), lambda i: (i, 0)),
            pl.BlockSpec((_BROW, C), lambda i: (i, 0)),
            pl.BlockSpec((_BROW, 1), lambda i: (i, 0)),
            pl.BlockSpec((1, C), lambda i: (0, 0)),
        ],
        out_specs=pl.BlockSpec((_BROW, C), lambda i: (i, 0)),
        out_shape=jax.ShapeDtypeStruct((N, C), jnp.float32),
    )(t0, t1, gs, dinv, b2.reshape(1, C))


def kernel(graph, data, W1, b1, W2, b2):
    src = graph[0]
    dst = graph[1]
    deg0, deg1 = _make_deg_kernel()(dst)
    dinv, xa, xb = _scale_split(deg0, deg1, data)
    sa, sb = _make_edge_sum(True, 192)(src, dst, xa, xb)
    gs = _dense(sa, sb, data, dinv, W1, b1, W2)
    t0, t1 = _make_edge_sum(False, 192)(src, dst, gs, gs)
    return _final(t0, t1, gs, dinv, b2)
